# baseline (speedup 1.0000x reference)
import jax
import jax.numpy as jnp
from jax.experimental import pallas as pl

N = 10000
E = 160000
D = 128
R = 51
L = 6
B = 256
K = 32
EPS = 1e-6


def _layer(x_in, rel, W, b, g, bb, src, dst, etype):
    msg = x_in[src] * rel[etype]
    deg = jax.ops.segment_sum(jnp.ones((E,), dtype=jnp.float32), dst, num_segments=N)
    denom = jnp.maximum(deg, 1.0)[:, None]
    mean = jax.ops.segment_sum(msg, dst, num_segments=N) / denom
    sq_mean = jax.ops.segment_sum(msg * msg, dst, num_segments=N) / denom
    mx = jax.ops.segment_max(msg, dst, num_segments=N)
    mn = jax.ops.segment_min(msg, dst, num_segments=N)
    has = (deg > 0)[:, None]
    mx = jnp.where(has, mx, 0.0)
    mn = jnp.where(has, mn, 0.0)
    std = jnp.sqrt(jnp.clip(sq_mean - mean * mean, EPS, None))
    features = jnp.concatenate([mean, mx, mn, std], axis=-1)
    scale = jnp.log(deg + 1.0)
    scale = scale / jnp.maximum(scale.mean(), EPS)
    scales = jnp.stack([jnp.ones_like(scale), scale, 1.0 / jnp.clip(scale, 1e-2, None)], axis=-1)
    update = (features[:, :, None] * scales[:, None, :]).reshape(N, 12 * D)
    out = jnp.concatenate([x_in, update], axis=-1) @ W + b
    mu = out.mean(axis=-1, keepdims=True)
    var = ((out - mu) ** 2).mean(axis=-1, keepdims=True)
    out = (out - mu) / jnp.sqrt(var + 1e-5) * g + bb
    return jax.nn.relu(out)


def _score_body(se_ref, re_ref, te_ref, o_ref):
    o_ref[...] = jnp.sum(se_ref[...] * re_ref[...] * te_ref[...], axis=-1, keepdims=True)


def kernel(x, edge_index, edge_type, h_index, r_index, t_index, params):
    src, dst = edge_index[0], edge_index[1]
    layer_input = x
    for i in range(L):
        hidden = _layer(layer_input, params["rel_%d" % i], params["W_%d" % i], params["b_%d" % i],
                        params["ln_g_%d" % i], params["ln_b_%d" % i], src, dst, edge_type)
        hidden = hidden + layer_input
        layer_input = hidden
    xf = layer_input
    se = xf[h_index.reshape(-1)]
    re = params["query"][r_index.reshape(-1)]
    te = xf[t_index.reshape(-1)]
    scores = pl.pallas_call(
        _score_body,
        out_shape=jax.ShapeDtypeStruct((B * K, 1), jnp.float32),
    )(se, re, te)
    return scores.reshape(B, K)
